# baseline (speedup 1.0000x reference)
import jax
import jax.numpy as jnp
from jax.experimental import pallas as pl
from jax.experimental.pallas import tpu as pltpu


def _copy_block(in_ref, out_ref):
    out_ref[...] = in_ref[...]


def kernel(x, pos_emb):
    seq_len = x.shape[1]
    d_model = pos_emb.shape[1]
    block_rows = 4096
    block_cols = 512
    grid = (pl.cdiv(seq_len, block_rows), pl.cdiv(d_model, block_cols))
    return pl.pallas_call(
        _copy_block,
        grid=grid,
        in_specs=[pl.BlockSpec((block_rows, block_cols), lambda i, j: (i, j))],
        out_specs=pl.BlockSpec((block_rows, block_cols), lambda i, j: (i, j)),
        out_shape=jax.ShapeDtypeStruct((seq_len, d_model), pos_emb.dtype),
    )(pos_emb)


# speedup vs baseline: 1.0544x; 1.0544x over previous
import jax
import jax.numpy as jnp
from jax.experimental import pallas as pl
from jax.experimental.pallas import tpu as pltpu


def _copy_block(in_ref, out_ref):
    out_ref[...] = in_ref[...]


def kernel(x, pos_emb):
    seq_len = x.shape[1]
    d_model = pos_emb.shape[1]
    block_rows = 3744
    num_blocks = pl.cdiv(seq_len, block_rows)
    return pl.pallas_call(
        _copy_block,
        grid=(num_blocks,),
        in_specs=[pl.BlockSpec((block_rows, d_model), lambda i: (i, 0))],
        out_specs=pl.BlockSpec((block_rows, d_model), lambda i: (i, 0)),
        out_shape=jax.ShapeDtypeStruct((seq_len, d_model), pos_emb.dtype),
    )(pos_emb)
